# Initial kernel scaffold; baseline (speedup 1.0000x reference)
import jax
import jax.numpy as jnp
from jax.experimental import pallas as pl


def kernel(x):
    raise NotImplementedError("write your pallas kernel here")


# speedup vs baseline: 9.2731x; 9.2731x over previous
import functools

import jax
import jax.numpy as jnp
from jax import lax
from jax.experimental import pallas as pl
from jax.experimental.pallas import tpu as pltpu
from jax.experimental.pallas import tpu_sc as plsc

ROWS = 128
N = 32768
KK = 256
L = 16
NCHUNK = N // L
NWORKERS = 32
ROWS_PER_W = ROWS // NWORKERS

MIN32 = -(2 ** 31)
M7F = 0x7FFFFFFF
LOW22 = 0x3FFFFF


def _key_of(v):
    b = lax.bitcast_convert_type(v, jnp.int32)
    return b ^ ((b >> 31) & M7F)


def _iota():
    return lax.iota(jnp.int32, 16)


def _topk_body(x_hbm, out_hbm, row_v, fine_v, coarse_v, ck_v, ci_v):
    cid = lax.axis_index("c")
    sid = lax.axis_index("s")
    wid = sid * 2 + cid

    zeros16 = jnp.zeros((L,), jnp.int32)
    ones16 = jnp.ones((L,), jnp.int32)
    iota16 = _iota()

    def per_row(r, _carry):
        row = wid * ROWS_PER_W + r
        pltpu.sync_copy(x_hbm.at[row], row_v)

        def clr_fine(i, c):
            fine_v[pl.ds(pl.multiple_of(i * L, L), L)] = zeros16
            return c

        lax.fori_loop(0, 1024, clr_fine, 0, unroll=8)

        def clr_coarse(i, c):
            coarse_v[pl.ds(pl.multiple_of(i * L, L), L)] = zeros16
            return c

        lax.fori_loop(0, 64, clr_coarse, 0, unroll=8)

        def hist(i, c):
            v = row_v[pl.ds(pl.multiple_of(i * L, L), L)]
            key = _key_of(v)
            ux = key ^ MIN32
            ubin = (ux >> 22) & jnp.int32(1023)
            plsc.addupdate_scatter(fine_v, [ubin * L + iota16], ones16)
            plsc.addupdate_scatter(coarse_v, [(ubin >> 4) * L + iota16], ones16)
            return c

        lax.fori_loop(0, NCHUNK, hist, 0, unroll=4)

        def cscan(j, st):
            acc, fb, fnd = st
            b = 63 - j
            v = coarse_v[pl.ds(pl.multiple_of(b * L, L), L)]
            s = jnp.sum(v)
            hit = jnp.logical_and(jnp.logical_not(fnd), acc + s >= KK)
            fb = jnp.where(hit, b, fb)
            fnd = jnp.logical_or(fnd, hit)
            acc = jnp.where(fnd, acc, acc + s)
            return acc, fb, fnd

        above_c, cstar, _ = lax.fori_loop(
            0, 64, cscan, (jnp.int32(0), jnp.int32(0), False))

        def fscan(j, st):
            acc, fb, fnd = st
            b = cstar * 16 + 15 - j
            v = fine_v[pl.ds(pl.multiple_of(b * L, L), L)]
            s = jnp.sum(v)
            hit = jnp.logical_and(jnp.logical_not(fnd), acc + s >= KK)
            fb = jnp.where(hit, b, fb)
            fnd = jnp.logical_or(fnd, hit)
            acc = jnp.where(fnd, acc, acc + s)
            return acc, fb, fnd

        above, b1, _ = lax.fori_loop(0, 16, fscan, (above_c, jnp.int32(0), False))
        k_rem = jnp.int32(KK) - above

        def comp(i, nc_vec):
            v = row_v[pl.ds(pl.multiple_of(i * L, L), L)]
            key = _key_of(v)
            ux = key ^ MIN32
            ubin = (ux >> 22) & jnp.int32(1023)
            m = ubin == b1
            pref = plsc.cumsum(m.astype(jnp.int32))
            pos = nc_vec + pref - 1
            plsc.store_scatter(ck_v, [pos], ux & LOW22, mask=m)
            plsc.store_scatter(ci_v, [pos], i * L + iota16, mask=m)
            return nc_vec + plsc.all_reduce_population_count(m)

        nc_vec = lax.fori_loop(0, NCHUNK, comp, zeros16, unroll=4)
        nc = jnp.max(nc_vec)
        nch = (nc + (L - 1)) // L

        def refine(bi, prefix):
            bit = jnp.int32(21) - bi
            cand = prefix | (jnp.int32(1) << bit)

            def cnt_chunk(j, cnt_v):
                base = pl.multiple_of(j * L, L)
                low = ck_v[pl.ds(base, L)]
                valid = (j * L + iota16) < nc
                m = jnp.logical_and(low >= cand, valid)
                return cnt_v + plsc.all_reduce_population_count(m)

            cnt = jnp.max(lax.fori_loop(0, nch, cnt_chunk, zeros16))
            return jnp.where(cnt >= k_rem, cand, prefix)

        prefix = lax.fori_loop(0, 22, refine, jnp.int32(0))

        def gt_chunk(j, cnt_v):
            base = pl.multiple_of(j * L, L)
            low = ck_v[pl.ds(base, L)]
            valid = (j * L + iota16) < nc
            m = jnp.logical_and(low > prefix, valid)
            return cnt_v + plsc.all_reduce_population_count(m)

        c_gt = jnp.max(lax.fori_loop(0, nch, gt_chunk, zeros16))
        ties_needed = k_rem - c_gt

        t_u = (b1 << 22) | prefix
        t_s = t_u ^ MIN32
        tb_vec = jnp.full((L,), t_s, jnp.int32)
        tb_vec = tb_vec ^ ((tb_vec >> 31) & M7F)
        t_f_vec = lax.bitcast_convert_type(tb_vec, jnp.float32)

        def outp(i, c):
            base = pl.ds(pl.multiple_of(i * L, L), L)
            v = row_v[base]
            key = _key_of(v)
            row_v[base] = jnp.where(key > t_s, v, jnp.float32(0.0))
            return c

        lax.fori_loop(0, NCHUNK, outp, 0, unroll=4)

        def fixup(j, seq):
            base = pl.multiple_of(j * L, L)
            low = ck_v[pl.ds(base, L)]
            idx = ci_v[pl.ds(base, L)]
            valid = (j * L + iota16) < nc
            m_eq = jnp.logical_and(low == prefix, valid)
            pref = plsc.cumsum(m_eq.astype(jnp.int32))
            keep = jnp.logical_and(m_eq, (seq + pref) <= ties_needed)
            plsc.store_scatter(row_v, [idx], t_f_vec, mask=keep)
            return seq + jnp.sum(m_eq.astype(jnp.int32))

        lax.fori_loop(0, nch, fixup, jnp.int32(0))

        pltpu.sync_copy(row_v, out_hbm.at[row])
        return _carry

    lax.fori_loop(0, ROWS_PER_W, per_row, 0)


@functools.partial(
    pl.kernel,
    out_type=jax.ShapeDtypeStruct((ROWS, N), jnp.float32),
    mesh=plsc.VectorSubcoreMesh(core_axis_name="c", subcore_axis_name="s"),
    compiler_params=pltpu.CompilerParams(needs_layout_passes=False),
    scratch_types=[
        pltpu.VMEM((N,), jnp.float32),
        pltpu.VMEM((1024 * L,), jnp.int32),
        pltpu.VMEM((64 * L,), jnp.int32),
        pltpu.VMEM((N,), jnp.int32),
        pltpu.VMEM((N,), jnp.int32),
    ],
)
def _topk_sc(x_hbm, out_hbm, row_v, fine_v, coarse_v, ck_v, ci_v):
    _topk_body(x_hbm, out_hbm, row_v, fine_v, coarse_v, ck_v, ci_v)


def kernel(x):
    return _topk_sc(x)


# speedup vs baseline: 25.2418x; 2.7220x over previous
import functools

import jax
import jax.numpy as jnp
from jax import lax
from jax.experimental import pallas as pl
from jax.experimental.pallas import tpu as pltpu
from jax.experimental.pallas import tpu_sc as plsc

ROWS = 128
N = 32768
KK = 256
L = 16
NCHUNK = N // L
NWORKERS = 32
ROWS_PER_W = ROWS // NWORKERS
FBINS = 1024
NGROUPS = FBINS // L

MIN32 = -(2 ** 31)
M7F = 0x7FFFFFFF
LOW22 = 0x3FFFFF


def _key_of(v):
    b = lax.bitcast_convert_type(v, jnp.int32)
    return b ^ ((b >> 31) & M7F)


def _topk_body(x_hbm, out_hbm, row_v, fine_v, gsum_v, ck_v, ci_v):
    cid = lax.axis_index("c")
    sid = lax.axis_index("s")
    wid = sid * 2 + cid

    zeros16 = jnp.zeros((L,), jnp.int32)
    ones16 = jnp.ones((L,), jnp.int32)
    iota16 = lax.iota(jnp.int32, L)
    lane_off = iota16 * FBINS
    lane15 = iota16 == (L - 1)

    def per_row(r, _carry):
        row = wid * ROWS_PER_W + r
        pltpu.sync_copy(x_hbm.at[row], row_v)

        @plsc.parallel_loop(0, FBINS, unroll=8)
        def _clear(i):
            fine_v[pl.ds(pl.multiple_of(i * L, L), L)] = zeros16

        @plsc.parallel_loop(0, NCHUNK, unroll=8)
        def _hist(i):
            v = row_v[pl.ds(pl.multiple_of(i * L, L), L)]
            key = _key_of(v)
            ubin = ((key ^ MIN32) >> 22) & (FBINS - 1)
            plsc.addupdate_scatter(fine_v, [lane_off + ubin], ones16)

        @plsc.parallel_loop(0, NGROUPS, unroll=2)
        def _gsum(g):
            base = g * L
            vec = fine_v[pl.ds(pl.multiple_of(base, L), L)]
            for l in range(1, L):
                vec = vec + fine_v[pl.ds(pl.multiple_of(l * FBINS + base, L), L)]
            cum = plsc.cumsum(vec)
            gidx = jnp.full((L,), g, jnp.int32)
            plsc.store_scatter(gsum_v, [gidx], cum, mask=lane15)

        gvecs = [gsum_v[pl.ds(k * L, L)] for k in range(NGROUPS // L)]
        sufs = [jnp.flip(plsc.cumsum(jnp.flip(g, 0)), 0) for g in gvecs]
        tots = [s[0] for s in sufs]
        cnt_g = zeros16
        hi = jnp.int32(0)
        for k in range(NGROUPS // L - 1, -1, -1):
            cnt_g = cnt_g + plsc.all_reduce_population_count((sufs[k] + hi) >= KK)
            hi = hi + tots[k]
        gstar = jnp.max(cnt_g) - 1
        acc_above = jnp.int32(0)
        for k in range(NGROUPS // L):
            acc_above = acc_above + jnp.sum(
                jnp.where((k * L + iota16) > gstar, gvecs[k], 0))

        base = gstar * L
        vec = fine_v[pl.ds(pl.multiple_of(base, L), L)]
        for l in range(1, L):
            vec = vec + fine_v[pl.ds(pl.multiple_of(l * FBINS + base, L), L)]
        suf = jnp.flip(plsc.cumsum(jnp.flip(vec, 0)), 0)
        cond = (acc_above + suf) >= KK
        cntv = plsc.all_reduce_population_count(cond)
        posv = cntv - 1
        b1v = gstar * L + posv
        count_above = acc_above + jnp.sum(jnp.where(iota16 > posv, vec, 0))
        k_rem = jnp.int32(KK) - count_above
        b1s = gstar * L + jnp.max(cntv) - 1

        @plsc.parallel_loop(0, NCHUNK, unroll=4, carry=zeros16)
        def _comp(i, nc_vec):
            v = row_v[pl.ds(pl.multiple_of(i * L, L), L)]
            key = _key_of(v)
            ux = key ^ MIN32
            ubin = (ux >> 22) & (FBINS - 1)
            m = ubin == b1v
            pref = plsc.cumsum(m.astype(jnp.int32))
            pos = nc_vec + pref - 1
            plsc.store_scatter(ck_v, [pos], ux & LOW22, mask=m)
            plsc.store_scatter(ci_v, [pos], i * L + iota16, mask=m)
            return nc_vec + plsc.all_reduce_population_count(m)

        nc = jnp.max(_comp)
        nch = (nc + (L - 1)) // L

        def refine(bi, prefix):
            bit = jnp.int32(21) - bi
            cand = prefix | (jnp.int32(1) << bit)

            def cnt_chunk(j, cnt_v):
                low = ck_v[pl.ds(pl.multiple_of(j * L, L), L)]
                valid = (j * L + iota16) < nc
                m = jnp.logical_and(low >= cand, valid)
                return cnt_v + plsc.all_reduce_population_count(m)

            cnt = jnp.max(lax.fori_loop(0, nch, cnt_chunk, zeros16))
            return jnp.where(cnt >= k_rem, cand, prefix)

        prefix = lax.fori_loop(0, 22, refine, jnp.int32(0))

        def gt_chunk(j, cnt_v):
            low = ck_v[pl.ds(pl.multiple_of(j * L, L), L)]
            valid = (j * L + iota16) < nc
            m = jnp.logical_and(low > prefix, valid)
            return cnt_v + plsc.all_reduce_population_count(m)

        c_gt = jnp.max(lax.fori_loop(0, nch, gt_chunk, zeros16))
        ties_needed = k_rem - c_gt

        t_u = (b1s << 22) | prefix
        t_s = t_u ^ MIN32
        tb_vec = jnp.full((L,), t_s, jnp.int32)
        tb_vec = tb_vec ^ ((tb_vec >> 31) & M7F)
        t_f_vec = lax.bitcast_convert_type(tb_vec, jnp.float32)

        @plsc.parallel_loop(0, NCHUNK, unroll=8)
        def _outp(i):
            sl = pl.ds(pl.multiple_of(i * L, L), L)
            v = row_v[sl]
            key = _key_of(v)
            row_v[sl] = jnp.where(key > t_s, v, jnp.float32(0.0))

        def fixup(j, seq):
            sl = pl.ds(pl.multiple_of(j * L, L), L)
            low = ck_v[sl]
            idx = ci_v[sl]
            valid = (j * L + iota16) < nc
            m_eq = jnp.logical_and(low == prefix, valid)
            pref = plsc.cumsum(m_eq.astype(jnp.int32))
            keep = jnp.logical_and(m_eq, (seq + pref) <= ties_needed)
            plsc.store_scatter(row_v, [idx], t_f_vec, mask=keep)
            return seq + jnp.sum(m_eq.astype(jnp.int32))

        lax.fori_loop(0, nch, fixup, jnp.int32(0))

        pltpu.sync_copy(row_v, out_hbm.at[row])
        return _carry

    lax.fori_loop(0, ROWS_PER_W, per_row, 0)


@functools.partial(
    pl.kernel,
    out_type=jax.ShapeDtypeStruct((ROWS, N), jnp.float32),
    mesh=plsc.VectorSubcoreMesh(core_axis_name="c", subcore_axis_name="s"),
    compiler_params=pltpu.CompilerParams(needs_layout_passes=False),
    scratch_types=[
        pltpu.VMEM((N,), jnp.float32),
        pltpu.VMEM((L * FBINS,), jnp.int32),
        pltpu.VMEM((NGROUPS,), jnp.int32),
        pltpu.VMEM((N,), jnp.int32),
        pltpu.VMEM((N,), jnp.int32),
    ],
)
def _topk_sc(x_hbm, out_hbm, row_v, fine_v, gsum_v, ck_v, ci_v):
    _topk_body(x_hbm, out_hbm, row_v, fine_v, gsum_v, ck_v, ci_v)


def kernel(x):
    return _topk_sc(x)
